# Initial kernel scaffold; baseline (speedup 1.0000x reference)
import jax
import jax.numpy as jnp
from jax.experimental import pallas as pl


def kernel(x, freqs_cis, wq, wk, wv, wo, q_norm_w, q_norm_b, k_norm_w, k_norm_b, attn_norm_w, attn_norm_b, ffn_norm_w, ffn_norm_b, router_w, router_b, w1, w2, w3):
    raise NotImplementedError("write your pallas kernel here")


# speedup vs baseline: 1.1605x; 1.1605x over previous
import functools

import jax
import jax.numpy as jnp
from jax.experimental import pallas as pl
from jax.experimental.pallas import tpu as pltpu

S, DIM, E, H = 2048, 768, 8, 2048
NEG_INF = -1e30


def _ln(x, w, b, eps=1e-5):
    mu = jnp.mean(x, axis=-1, keepdims=True)
    var = jnp.mean((x - mu) ** 2, axis=-1, keepdims=True)
    return (x - mu) / jnp.sqrt(var + eps) * w + b


def _ka_body(x_ref, an_ref, xn_ref):
    xn_ref[...] = _ln(x_ref[...], an_ref[0, :], an_ref[1, :])


def _kb_body(xn_ref, wq_ref, wk_ref, wv_ref, qn_ref, kn_ref,
             q_ref, k_ref, v_ref):
    xn = xn_ref[...]
    q_ref[...] = _ln(jnp.dot(xn, wq_ref[...].T,
                             preferred_element_type=jnp.float32),
                     qn_ref[0, :], qn_ref[1, :])
    k_ref[...] = _ln(jnp.dot(xn, wk_ref[...].T,
                             preferred_element_type=jnp.float32),
                     kn_ref[0, :], kn_ref[1, :])
    v_ref[...] = jnp.dot(xn, wv_ref[...].T, preferred_element_type=jnp.float32)


def _kc_body(q_ref, k_ref, aw_ref):
    a = jax.lax.dot_general(q_ref[...], k_ref[...], (((0,), (0,)), ((), ())),
                            preferred_element_type=jnp.float32)
    a = a - jnp.max(a, axis=-1, keepdims=True)
    ea = jnp.exp(a)
    aw_ref[...] = ea / jnp.sum(ea, axis=-1, keepdims=True)


def _kd_body(v_ref, aw_ref, attn_ref):
    attn_ref[...] = jnp.dot(v_ref[...], aw_ref[...].T,
                            preferred_element_type=jnp.float32)


def _ke_body(x_ref, attn_ref, wo_ref, fn_ref, hn_ref):
    h = x_ref[...] + jnp.dot(attn_ref[...], wo_ref[...].T,
                             preferred_element_type=jnp.float32)
    hn_ref[...] = _ln(h, fn_ref[0, :], fn_ref[1, :])


def _kf_body(hn_ref, rw_ref, rb_ref, route_ref, aux_ref):
    lg = jnp.dot(hn_ref[...], rw_ref[...].T,
                 preferred_element_type=jnp.float32) + rb_ref[0, :]
    norm = jnp.sqrt(jnp.sum(lg * lg, axis=0, keepdims=True))
    lg = lg / jnp.maximum(norm, 1e-12)
    lg = lg - jnp.max(lg, axis=-1, keepdims=True)
    el = jnp.exp(lg)
    probs = el / jnp.sum(el, axis=-1, keepdims=True)
    aux_ref[...] = jnp.sum((1.0 / E - probs) ** 2, keepdims=True)

    e_iota = jax.lax.broadcasted_iota(jnp.int32, (S, E), 1)
    m1 = jnp.max(probs, axis=-1, keepdims=True)
    i1 = jnp.min(jnp.where(probs == m1, e_iota, E), axis=-1, keepdims=True)
    pm = jnp.where(e_iota == i1, NEG_INF, probs)
    m2 = jnp.max(pm, axis=-1, keepdims=True)
    i2 = jnp.min(jnp.where(pm == m2, e_iota, E), axis=-1, keepdims=True)
    route_ref[...] = jnp.concatenate(
        [i1.astype(jnp.float32), i2.astype(jnp.float32), m1, m2], axis=-1)


def _k5_body(hn_ref, w1_ref, w2_ref, w3_ref, route_ref, out_ref, acc_ref):
    e = pl.program_id(0)
    s = pl.program_id(1)
    ts = hn_ref.shape[0]
    xb = hn_ref[...]
    h1 = jnp.dot(xb, w1_ref[0].T, preferred_element_type=jnp.float32)
    h3 = jnp.dot(xb, w3_ref[0].T, preferred_element_type=jnp.float32)
    hh = jnp.sin(h1) * h3
    y = jnp.dot(hh, w2_ref[0].T, preferred_element_type=jnp.float32)
    r = route_ref[...]
    ef = jnp.float32(e)
    gate = (jnp.where(r[:, 0:1] == ef, r[:, 2:3], 0.0)
            + jnp.where(r[:, 1:2] == ef, r[:, 3:4], 0.0))
    contrib = gate * y

    @pl.when(e == 0)
    def _():
        acc_ref[pl.ds(s * ts, ts), :] = contrib

    @pl.when(e != 0)
    def _():
        acc_ref[pl.ds(s * ts, ts), :] += contrib

    out_ref[...] = acc_ref[pl.ds(s * ts, ts), :]


def _stages(x2, wq, wk, wv, wo, an, qn, kn, fn, router_w, rb, w1, w2, w3):
    f32 = jnp.float32

    xn = pl.pallas_call(
        _ka_body,
        out_shape=jax.ShapeDtypeStruct((S, DIM), f32),
    )(x2, an)

    q, k, v = pl.pallas_call(
        _kb_body,
        out_shape=[jax.ShapeDtypeStruct((S, DIM), f32)] * 3,
    )(xn, wq, wk, wv, qn, kn)

    aw = pl.pallas_call(
        _kc_body,
        out_shape=jax.ShapeDtypeStruct((DIM, DIM), f32),
    )(q, k)

    attn = pl.pallas_call(
        _kd_body,
        out_shape=jax.ShapeDtypeStruct((S, DIM), f32),
    )(v, aw)

    hn = pl.pallas_call(
        _ke_body,
        out_shape=jax.ShapeDtypeStruct((S, DIM), f32),
    )(x2, attn, wo, fn)

    route, aux = pl.pallas_call(
        _kf_body,
        out_shape=[
            jax.ShapeDtypeStruct((S, 4), f32),
            jax.ShapeDtypeStruct((1, 1), f32),
        ],
    )(hn, router_w, rb)

    TS5 = 256
    g5 = S // TS5
    out = pl.pallas_call(
        _k5_body,
        grid=(E, g5),
        in_specs=[
            pl.BlockSpec((TS5, DIM), lambda e, s: (s, 0)),
            pl.BlockSpec((1, H, DIM), lambda e, s: (e, 0, 0)),
            pl.BlockSpec((1, DIM, H), lambda e, s: (e, 0, 0)),
            pl.BlockSpec((1, H, DIM), lambda e, s: (e, 0, 0)),
            pl.BlockSpec((TS5, 4), lambda e, s: (s, 0)),
        ],
        out_specs=pl.BlockSpec((TS5, DIM), lambda e, s: (s, 0)),
        out_shape=jax.ShapeDtypeStruct((S, DIM), f32),
        scratch_shapes=[pltpu.VMEM((S, DIM), f32)],
        compiler_params=pltpu.CompilerParams(
            dimension_semantics=("arbitrary", "arbitrary")),
    )(hn, w1, w2, w3, route)

    return v, aw, hn, None, route, aux, out


def kernel(x, freqs_cis, wq, wk, wv, wo, q_norm_w, q_norm_b, k_norm_w,
           k_norm_b, attn_norm_w, attn_norm_b, ffn_norm_w, ffn_norm_b,
           router_w, router_b, w1, w2, w3):
    del freqs_cis
    x2 = x.reshape(S, DIM)
    an = jnp.stack([attn_norm_w, attn_norm_b])
    qn = jnp.stack([q_norm_w, q_norm_b])
    kn = jnp.stack([k_norm_w, k_norm_b])
    fn = jnp.stack([ffn_norm_w, ffn_norm_b])
    rb = router_b.reshape(1, E)
    *_rest, aux, out = _stages(x2, wq, wk, wv, wo, an, qn, kn, fn,
                               router_w, rb, w1, w2, w3)
    return out.reshape(1, S, DIM), aux[0, 0]
